# Initial kernel scaffold; baseline (speedup 1.0000x reference)
import jax
import jax.numpy as jnp
from jax.experimental import pallas as pl


def kernel(pos, edge_index, edge_attr, batch, en0_Win, en0_bin, en0_Wsh, en0_bsh, en0_Wout, en0_bout, root0_W, root0_b, en1_Win, en1_bin, en1_Wsh, en1_bsh, en1_Wout, en1_bout, root1_W, root1_b, en2_Win, en2_bin, en2_Wsh, en2_bsh, en2_Wout, en2_bout, root2_W, root2_b, proj_W, proj_b):
    raise NotImplementedError("write your pallas kernel here")


# speedup vs baseline: 5.0312x; 5.0312x over previous
import functools

import jax
import jax.numpy as jnp
from jax import lax
from jax.experimental import pallas as pl
from jax.experimental.pallas import tpu as pltpu
from jax.experimental.pallas import tpu_sc as plsc

_EPS = 1e-2
_N = 10000
_NP = 10240
_E = 160000
_EP = 163840
_G = 64
_NC, _NS = 2, 16
_NW = _NC * _NS
_CPW = _EP // _NW
_CHUNK = 128
_NCH = _CPW // _CHUNK
_RPT = _NP // _NS

_f32 = jnp.float32


def _silu(v):
    return v / (1.0 + jnp.exp(-v))


def _sc_gather(x, src, dst):
    mesh = plsc.VectorSubcoreMesh(core_axis_name="c", subcore_axis_name="s")
    folded = jax.ShapeDtypeStruct((_EP // 8, 128), _f32)

    @functools.partial(
        pl.kernel,
        out_type=(folded, folded),
        mesh=mesh,
        compiler_params=pltpu.CompilerParams(use_tc_tiling_on_sc=False),
        scratch_types=[
            pltpu.VMEM((_CPW,), jnp.int32),
            pltpu.VMEM((_CPW,), jnp.int32),
            pltpu.VMEM((_CHUNK, 16), _f32),
            pltpu.VMEM((_CHUNK, 16), _f32),
            pltpu.VMEM((16, 128), _f32),
            pltpu.VMEM((16, 128), _f32),
            pltpu.SemaphoreType.DMA,
            pltpu.SemaphoreType.DMA,
        ],
    )
    def k(x_hbm, s_hbm, d_hbm, xsf_hbm, xdf_hbm,
          sv, dv, r0, r1, p0, p1, sem0, sem1):
        wid = lax.axis_index("s") * _NC + lax.axis_index("c")
        base = wid * _CPW
        pltpu.sync_copy(s_hbm.at[pl.ds(base, _CPW)], sv)
        pltpu.sync_copy(d_hbm.at[pl.ds(base, _CPW)], dv)

        def body(c, carry):
            off = c * _CHUNK
            a = pltpu.async_copy(x_hbm.at[sv.at[pl.ds(off, _CHUNK)]], r0, sem0)
            b = pltpu.async_copy(x_hbm.at[dv.at[pl.ds(off, _CHUNK)]], r1, sem1)
            a.wait()
            b.wait()
            for j in range(16):
                for m in range(8):
                    p0[j, pl.ds(16 * m, 16)] = r0[8 * j + m]
                    p1[j, pl.ds(16 * m, 16)] = r1[8 * j + m]
            roff = (base + off) // 8
            pltpu.sync_copy(p0, xsf_hbm.at[pl.ds(roff, 16)])
            pltpu.sync_copy(p1, xdf_hbm.at[pl.ds(roff, 16)])
            return carry

        lax.fori_loop(0, _NCH, body, 0)

    return k(x, src, dst)


def _sc_scatter(msg_f, dst, zero_rows):
    mesh = plsc.VectorSubcoreMesh(core_axis_name="c", subcore_axis_name="s")

    @functools.partial(
        pl.kernel,
        out_type=jax.ShapeDtypeStruct((_NC, _NP, 16), _f32),
        mesh=mesh,
        compiler_params=pltpu.CompilerParams(use_tc_tiling_on_sc=False),
        scratch_types=[
            pltpu.VMEM((_CHUNK, 16), _f32),
            pltpu.VMEM((16, 128), _f32),
            pltpu.VMEM((_CHUNK,), jnp.int32),
            pltpu.VMEM((_RPT, 16), _f32),
            pltpu.VMEM_SHARED((_NP, 16), _f32),
        ],
    )
    def k(mf_hbm, d_hbm, z_hbm, out_hbm, mv, pv, iv, bv, shared):
        cid = lax.axis_index("c")
        sid = lax.axis_index("s")
        base = (sid * _NC + cid) * _CPW
        rows = pl.ds(sid * _RPT, _RPT)
        pltpu.sync_copy(z_hbm.at[pl.ds(0, _RPT)], bv)
        pltpu.sync_copy(bv, shared.at[rows])
        plsc.subcore_barrier()

        def body(c, carry):
            off = base + c * _CHUNK
            pltpu.sync_copy(mf_hbm.at[pl.ds(off // 8, 16)], pv)
            for j in range(16):
                for m in range(8):
                    mv[8 * j + m] = pv[j, pl.ds(16 * m, 16)]
            pltpu.sync_copy(d_hbm.at[pl.ds(off, _CHUNK)], iv)
            pltpu.sync_copy(mv, shared.at[iv], add=True)
            return carry

        lax.fori_loop(0, _NCH, body, 0)
        plsc.subcore_barrier()
        pltpu.sync_copy(shared.at[rows], bv)
        pltpu.sync_copy(bv, out_hbm.at[cid, rows])

    return k(msg_f, dst, zero_rows)


def _static_selectors():
    import numpy as np
    selp = np.zeros((128, 16), np.float32)
    pair = np.zeros((16, 8), np.float32)
    bc2 = np.zeros((8, 16), np.float32)
    A = np.zeros((80, 128), np.float32)
    B = np.zeros((32, 128), np.float32)
    offs2 = [0, 2, 8, 10, 12]
    for a in range(8):
        for c in range(2):
            selp[16 * a + c, 2 * a + c] = 1.0
            pair[2 * a + c, a] = 1.0
            bc2[a, 2 * a + c] = 1.0
            for f in range(5):
                A[16 * f + 2 * a + c, 16 * a + offs2[f] + c] = 1.0
        for g in range(4):
            B[8 * g + a, 16 * a + 4 + g] = 1.0
    return (jnp.asarray(selp), jnp.asarray(pair), jnp.asarray(bc2),
            jnp.asarray(A), jnp.asarray(B))


def _tc_static(xs_f, xd_f, ea_f):
    be = 16384
    bf = be // 8
    grid = (_EP // be,)
    selp, pair, bc2, A, B = _static_selectors()

    def body(xs_ref, xd_ref, ea_ref, sp_ref, pr_ref, bc_ref, a_ref, b_ref,
             o_ref):
        dot = lambda u, v: jnp.dot(u, v, preferred_element_type=_f32)
        ps = dot(xs_ref[...], sp_ref[...])
        pd = dot(xd_ref[...], sp_ref[...])
        ea = ea_ref[...]
        vec = pd - ps
        nsq = dot(vec * vec, pr_ref[...])
        vn = jnp.sqrt(nsq)
        unit = vec / (dot(vn, bc_ref[...]) + _EPS)
        two = jnp.concatenate(
            [ea, unit, 1.0 / (ea + _EPS), ea * ea,
             1.0 / (ea * ea + _EPS)], axis=1)
        one = jnp.concatenate(
            [vn, 1.0 / (vn + _EPS), nsq, 1.0 / (nsq + _EPS)], axis=1)
        o_ref[...] = dot(two, a_ref[...]) + dot(one, b_ref[...])

    return pl.pallas_call(
        body,
        grid=grid,
        in_specs=[
            pl.BlockSpec((bf, 128), lambda i: (i, 0)),
            pl.BlockSpec((bf, 128), lambda i: (i, 0)),
            pl.BlockSpec((bf, 16), lambda i: (i, 0)),
            pl.BlockSpec((128, 16), lambda i: (0, 0)),
            pl.BlockSpec((16, 8), lambda i: (0, 0)),
            pl.BlockSpec((8, 16), lambda i: (0, 0)),
            pl.BlockSpec((80, 128), lambda i: (0, 0)),
            pl.BlockSpec((32, 128), lambda i: (0, 0)),
        ],
        out_specs=pl.BlockSpec((bf, 128), lambda i: (i, 0)),
        out_shape=jax.ShapeDtypeStruct((_EP // 8, 128), _f32),
    )(xs_f, xd_f, ea_f, selp, pair, bc2, A, B)


def _tc_mlp(ci, static_f, xs_f, xd_f, WinS, WinX, WinD, binf, Wshf, bshf,
            Wout2, bout2, R2):
    be = 16384
    bf = be // 8
    grid = (_EP // be,)
    wc = ci * 128

    bh = jnp.bfloat16

    def hidden_body(st_ref, xs_ref, xd_ref, ws_ref, wx_ref, wd_ref,
                    bi_ref, wsh_ref, bsh_ref, o_ref):
        h = (jnp.dot(st_ref[...], ws_ref[...], preferred_element_type=_f32)
             + jnp.dot(xs_ref[...], wx_ref[...], preferred_element_type=_f32)
             + jnp.dot(xd_ref[...], wd_ref[...], preferred_element_type=_f32)
             + bi_ref[...])
        h = _silu(h)
        for i in range(15):
            h = _silu(jnp.dot(h.astype(bh), wsh_ref[i],
                              preferred_element_type=_f32) + bsh_ref[i])
        o_ref[...] = h

    hf = pl.pallas_call(
        hidden_body,
        grid=grid,
        in_specs=[
            pl.BlockSpec((bf, 128), lambda i: (i, 0)),
            pl.BlockSpec((bf, 128), lambda i: (i, 0)),
            pl.BlockSpec((bf, 128), lambda i: (i, 0)),
            pl.BlockSpec((128, 128), lambda i: (0, 0)),
            pl.BlockSpec((128, 128), lambda i: (0, 0)),
            pl.BlockSpec((128, 128), lambda i: (0, 0)),
            pl.BlockSpec((1, 128), lambda i: (0, 0)),
            pl.BlockSpec((15, 128, 128), lambda i: (0, 0, 0)),
            pl.BlockSpec((15, 1, 128), lambda i: (0, 0, 0)),
        ],
        out_specs=pl.BlockSpec((bf, 128), lambda i: (i, 0)),
        out_shape=jax.ShapeDtypeStruct((_EP // 8, 128), _f32),
    )(static_f, xs_f, xd_f, WinS, WinX, WinD, binf,
      Wshf.astype(jnp.bfloat16), bshf)

    def msg_body(h_ref, xs_ref, wo_ref, bo_ref, r_ref, o_ref):
        we = (jnp.dot(h_ref[...], wo_ref[...], preferred_element_type=_f32)
              + bo_ref[...])
        xsb = jnp.dot(xs_ref[...], r_ref[...], preferred_element_type=_f32)
        mw = xsb * we
        w = wc
        while w > 128:
            w //= 2
            mw = mw[:, :w] + mw[:, w:]
        o_ref[...] = mw

    return pl.pallas_call(
        msg_body,
        grid=grid,
        in_specs=[
            pl.BlockSpec((bf, 128), lambda i: (i, 0)),
            pl.BlockSpec((bf, 128), lambda i: (i, 0)),
            pl.BlockSpec((128, wc), lambda i: (0, 0)),
            pl.BlockSpec((1, wc), lambda i: (0, 0)),
            pl.BlockSpec((128, wc), lambda i: (0, 0)),
        ],
        out_specs=pl.BlockSpec((bf, 128), lambda i: (i, 0)),
        out_shape=jax.ShapeDtypeStruct((_EP // 8, 128), _f32),
    )(hf, xs_f, Wout2, bout2, R2)


def _tc_update(agg, x, rootW, rootb, residual):
    def body(a_ref, x_ref, w_ref, b_ref, o_ref):
        x_b = x_ref[...]
        o = (a_ref[0] + a_ref[1]
             + jnp.dot(x_b, w_ref[...], preferred_element_type=_f32)
             + b_ref[...])
        if residual:
            o = o + x_b
        o_ref[...] = _silu(o)

    return pl.pallas_call(
        body,
        in_specs=[
            pl.BlockSpec((_NC, _NP, 16), lambda: (0, 0, 0)),
            pl.BlockSpec((_NP, 16), lambda: (0, 0)),
            pl.BlockSpec((16, 16), lambda: (0, 0)),
            pl.BlockSpec((1, 16), lambda: (0, 0)),
        ],
        out_specs=pl.BlockSpec((_NP, 16), lambda: (0, 0)),
        out_shape=jax.ShapeDtypeStruct((_NP, 16), _f32),
    )(agg, x, rootW, rootb)


def _tc_readout(x, batch2, batchT, proj_W, proj_b):
    def body(x_ref, b_ref, bt_ref, pw_ref, pb_ref, o_ref, cat_ref):
        x_b = x_ref[...]
        b_b = b_ref[...]
        gio = lax.broadcasted_iota(jnp.int32, (_G, _NP), 0)
        onehot = jnp.where(gio == bt_ref[...], 1.0, 0.0)
        s_all = jnp.dot(onehot, x_b, preferred_element_type=_f32)
        cnt_all = jnp.sum(onehot, axis=1, keepdims=True)
        mean_all = s_all / jnp.maximum(cnt_all, 1.0)

        def g_body(g, carry):
            m = b_b == g
            mx = jnp.max(jnp.where(m, x_b, -jnp.inf), axis=0, keepdims=True)
            cat_ref[pl.ds(g, 1), 32:48] = mx
            return carry

        lax.fori_loop(0, _G, g_body, 0)
        cat_ref[:, 0:16] = s_all
        cat_ref[:, 16:32] = mean_all
        cat = cat_ref[...]
        cat = jnp.where(jnp.broadcast_to(cnt_all > 0.0, (_G, 48)), cat, 0.0)
        o_ref[...] = (jnp.dot(cat, pw_ref[...],
                              preferred_element_type=_f32) + pb_ref[...])

    return pl.pallas_call(
        body,
        in_specs=[
            pl.BlockSpec((_NP, 16), lambda: (0, 0)),
            pl.BlockSpec((_NP, 1), lambda: (0, 0)),
            pl.BlockSpec((1, _NP), lambda: (0, 0)),
            pl.BlockSpec((48, 1), lambda: (0, 0)),
            pl.BlockSpec((1, 1), lambda: (0, 0)),
        ],
        out_specs=pl.BlockSpec((_G, 1), lambda: (0, 0)),
        out_shape=jax.ShapeDtypeStruct((_G, 1), _f32),
        scratch_shapes=[pltpu.VMEM((_G, 48), _f32)],
    )(x, batch2, batchT, proj_W, proj_b)


def _prep_layer_weights(ci, Win, bin_, Wsh, bsh, Wout, bout):
    g0 = 2 + 2 * ci
    eye8 = jnp.eye(8, dtype=_f32)
    wo = ci * 16
    WinS = jnp.concatenate(
        [Win[0:2], Win[g0:g0 + 12], jnp.zeros((2, 16), _f32)], axis=0)
    WinX = jnp.concatenate([Win[2:2 + ci],
                            jnp.zeros((16 - ci, 16), _f32)], axis=0)
    WinD = jnp.concatenate([Win[2 + ci:2 + 2 * ci],
                            jnp.zeros((16 - ci, 16), _f32)], axis=0)
    Wshf = jax.vmap(lambda w: jnp.kron(eye8, w))(Wsh)
    bshf = jnp.tile(bsh, (1, 8)).reshape(15, 1, 128)

    def imajor(m):
        return m.reshape(-1, 8, ci, 16).transpose(0, 2, 1, 3) \
                .reshape(-1, ci * 128)

    R = jnp.concatenate(
        [jnp.kron(jnp.eye(ci, dtype=_f32), jnp.ones((1, 16), _f32)),
         jnp.zeros((16 - ci, wo), _f32)], axis=0)
    Wout2 = imajor(jnp.kron(eye8, Wout))
    R2 = imajor(jnp.kron(eye8, R))
    bout2 = imajor(jnp.tile(bout, 8).reshape(1, 8 * wo))
    return (jnp.kron(eye8, WinS), jnp.kron(eye8, WinX), jnp.kron(eye8, WinD),
            jnp.tile(bin_, 8).reshape(1, 128), Wshf, bshf, Wout2, bout2, R2)


def kernel(pos, edge_index, edge_attr, batch,
           en0_Win, en0_bin, en0_Wsh, en0_bsh, en0_Wout, en0_bout,
           root0_W, root0_b,
           en1_Win, en1_bin, en1_Wsh, en1_bsh, en1_Wout, en1_bout,
           root1_W, root1_b,
           en2_Win, en2_bin, en2_Wsh, en2_bsh, en2_Wout, en2_bout,
           root2_W, root2_b,
           proj_W, proj_b):
    src = jnp.concatenate([edge_index[0],
                           jnp.zeros((_EP - _E,), jnp.int32)])
    dst = jnp.concatenate([edge_index[1],
                           jnp.full((_EP - _E,), _N, jnp.int32)])
    ea_f = jnp.pad(edge_attr,
                   ((0, _EP - _E), (0, 0))).reshape(_EP // 8, 16)
    batch2 = jnp.pad(batch, (0, _NP - _N),
                     constant_values=_G).reshape(_NP, 1)
    zero_rows = jnp.zeros((_RPT, 16), _f32)

    x = jnp.pad(pos, ((0, _NP - _N), (0, 14)))

    layers = [
        (2, en0_Win, en0_bin, en0_Wsh, en0_bsh, en0_Wout, en0_bout,
         root0_W, root0_b, False),
        (16, en1_Win, en1_bin, en1_Wsh, en1_bsh, en1_Wout, en1_bout,
         root1_W, root1_b, True),
        (16, en2_Win, en2_bin, en2_Wsh, en2_bsh, en2_Wout, en2_bout,
         root2_W, root2_b, True),
    ]

    static_f = None
    for ci, Win, bin_, Wsh, bsh, Wout, bout, rW, rb, res in layers:
        xs_f, xd_f = _sc_gather(x, src, dst)
        if static_f is None:
            static_f = _tc_static(xs_f, xd_f, ea_f)
        wts = _prep_layer_weights(ci, Win, bin_, Wsh, bsh, Wout, bout)
        msg_f = _tc_mlp(ci, static_f, xs_f, xd_f, *wts)
        agg = _sc_scatter(msg_f, dst, zero_rows)
        rWp = jnp.pad(rW, ((0, 16 - ci), (0, 0)))
        x = _tc_update(agg, x, rWp, rb.reshape(1, 16), res)

    y = _tc_readout(x, batch2, batch2.reshape(1, _NP), proj_W,
                    proj_b.reshape(1, 1))
    return y.reshape(-1)
